# baseline (speedup 1.0000x reference)
import jax
import jax.numpy as jnp
from jax import lax
from jax.experimental import pallas as pl
from jax.experimental.pallas import tpu as pltpu
from jax.experimental.pallas import tpu_sc as plsc

B = 16384
L = 16
NW = 32
BPW = B // NW
ROWS_PW = 4 * BPW
NCHUNK = ROWS_PW // 128


def _body(op1_hbm, op2_hbm, res_hbm, tbl_hbm, out_hbm,
          i1_v, i2_v, i3_v, comb_v, rows_v, sem, sem2):
    nc = 2
    wid = lax.axis_index("s") * nc + lax.axis_index("c")
    base = wid * BPW

    pltpu.sync_copy(op1_hbm.at[pl.ds(base, BPW)], i1_v)
    pltpu.sync_copy(op2_hbm.at[pl.ds(base, BPW)], i2_v)
    pltpu.sync_copy(res_hbm.at[pl.ds(base, BPW)], i3_v)

    lane = lax.iota(jnp.int32, L)
    for g in range(BPW // L):
        a1 = i1_v[pl.ds(g * L, L)]
        a2 = i2_v[pl.ds(g * L, L)]
        a3 = i3_v[pl.ds(g * L, L)]
        pos0 = 64 * g + 4 * lane
        plsc.store_scatter(comb_v, [pos0], a1)
        plsc.store_scatter(comb_v, [pos0 + 1], a2 + 100)
        plsc.store_scatter(comb_v, [pos0 + 2], 2 * a3 + 200)
        plsc.store_scatter(comb_v, [pos0 + 3], 2 * a3 + 201)

    gathers = []
    for j in range(NCHUNK):
        gathers.append(pltpu.async_copy(
            tbl_hbm.at[comb_v.at[pl.ds(j * 128, 128)]],
            rows_v.at[pl.ds(j * 128, 128)],
            sem))
    stores = []
    obase = wid * ROWS_PW
    for j in range(NCHUNK):
        gathers[j].wait()
        stores.append(pltpu.async_copy(
            rows_v.at[pl.ds(j * 128, 128)],
            out_hbm.at[pl.ds(obase + j * 128, 128)],
            sem2))
    for s in stores:
        s.wait()


def kernel(op1, op2, res, W_op1, W_op2, W_sum):
    tbl = jnp.concatenate(
        [W_op1, W_op2, W_sum.reshape(-1, W_op1.shape[1])], axis=0)

    mesh = plsc.VectorSubcoreMesh(core_axis_name="c", subcore_axis_name="s")
    out = pl.kernel(
        _body,
        out_type=jax.ShapeDtypeStruct((4 * B, 32), jnp.float32),
        mesh=mesh,
        scratch_types=[
            pltpu.VMEM((BPW,), jnp.int32),
            pltpu.VMEM((BPW,), jnp.int32),
            pltpu.VMEM((BPW,), jnp.int32),
            pltpu.VMEM((ROWS_PW,), jnp.int32),
            pltpu.VMEM((ROWS_PW, 32), jnp.float32),
            pltpu.SemaphoreType.DMA,
            pltpu.SemaphoreType.DMA,
        ],
        compiler_params=pltpu.CompilerParams(
            needs_layout_passes=False, use_tc_tiling_on_sc=False),
    )(op1.astype(jnp.int32), op2.astype(jnp.int32), res.astype(jnp.int32),
      tbl)
    return out.reshape(B, 128)


# speedup vs baseline: 1.4671x; 1.4671x over previous
import jax
import jax.numpy as jnp
from jax import lax
from jax.experimental import pallas as pl
from jax.experimental.pallas import tpu as pltpu
from jax.experimental.pallas import tpu_sc as plsc

B = 16384
L = 16
NW = 32
BPW = B // NW
ROWS_PW = 4 * BPW
NCHUNK = ROWS_PW // 128


def _body(op1_hbm, op2_hbm, res_hbm, tbl_hbm, out_hbm,
          i1_v, i2_v, i3_v, comb_v, rows_v, tbl_v, sem, sem2):
    nc = 2
    wid = lax.axis_index("s") * nc + lax.axis_index("c")
    base = wid * BPW

    @pl.when(lax.axis_index("s") == 0)
    def _():
        pltpu.sync_copy(tbl_hbm, tbl_v)
    pltpu.sync_copy(op1_hbm.at[pl.ds(base, BPW)], i1_v)
    pltpu.sync_copy(op2_hbm.at[pl.ds(base, BPW)], i2_v)
    pltpu.sync_copy(res_hbm.at[pl.ds(base, BPW)], i3_v)

    lane = lax.iota(jnp.int32, L)
    for g in range(BPW // L):
        a1 = i1_v[pl.ds(g * L, L)]
        a2 = i2_v[pl.ds(g * L, L)]
        a3 = i3_v[pl.ds(g * L, L)]
        pos0 = 64 * g + 4 * lane
        plsc.store_scatter(comb_v, [pos0], a1)
        plsc.store_scatter(comb_v, [pos0 + 1], a2 + 100)
        plsc.store_scatter(comb_v, [pos0 + 2], 2 * a3 + 200)
        plsc.store_scatter(comb_v, [pos0 + 3], 2 * a3 + 201)
    plsc.subcore_barrier()

    gathers = []
    for j in range(NCHUNK):
        gathers.append(pltpu.async_copy(
            tbl_v.at[comb_v.at[pl.ds(j * 128, 128)]],
            rows_v.at[pl.ds(j * 128, 128)],
            sem))
    for g in gathers:
        g.wait()

    pltpu.sync_copy(rows_v, out_hbm.at[pl.ds(wid * ROWS_PW, ROWS_PW)])


def kernel(op1, op2, res, W_op1, W_op2, W_sum):
    tbl = jnp.concatenate(
        [W_op1, W_op2, W_sum.reshape(-1, W_op1.shape[1])], axis=0)

    mesh = plsc.VectorSubcoreMesh(core_axis_name="c", subcore_axis_name="s")
    out = pl.kernel(
        _body,
        out_type=jax.ShapeDtypeStruct((4 * B, 32), jnp.float32),
        mesh=mesh,
        scratch_types=[
            pltpu.VMEM((BPW,), jnp.int32),
            pltpu.VMEM((BPW,), jnp.int32),
            pltpu.VMEM((BPW,), jnp.int32),
            pltpu.VMEM((ROWS_PW,), jnp.int32),
            pltpu.VMEM((ROWS_PW, 32), jnp.float32),
            pltpu.VMEM_SHARED((600, 32), jnp.float32),
            pltpu.SemaphoreType.DMA,
            pltpu.SemaphoreType.DMA,
        ],
        compiler_params=pltpu.CompilerParams(
            needs_layout_passes=False, use_tc_tiling_on_sc=False),
    )(op1.astype(jnp.int32), op2.astype(jnp.int32), res.astype(jnp.int32),
      tbl)
    return out.reshape(B, 128)
